# baseline (speedup 1.0000x reference)
import functools
import jax
import jax.numpy as jnp
from jax import lax
from jax.experimental import pallas as pl
from jax.experimental.pallas import tpu as pltpu

B, T, D, NT = 1024, 200, 128, 20000
BB = 32
NBLK = B // BB
NTB = -(-T // BB)
FULLTB = T // BB
TROWS = NTB * BB * T


def _dense_body(code_ref, num_ref, mask_ref, tid_ref, age_ref, gen_ref,
                age_t, gen_t, typ_t, cls_r, sep_r, mcc_r, text_ref, out_ref):
    i = pl.program_id(0)
    code = code_ref[...]
    num = num_ref[...]
    m = mask_ref[...]
    mcc = mcc_r[...].reshape(1, 1, D)
    num = jnp.where(m != 0.0, mcc, num)
    tid = tid_ref[...]
    typ = typ_t[...]
    te = jnp.where(tid == 0, typ[0].reshape(1, 1, D),
         jnp.where(tid == 1, typ[1].reshape(1, 1, D),
         jnp.where(tid == 2, typ[2].reshape(1, 1, D),
                   typ[3].reshape(1, 1, D))))
    tab = code + num + te

    sep = sep_r[...].reshape(1, 1, D)
    sep_b = jnp.broadcast_to(sep, (BB, T, D))

    @pl.when(i < FULLTB)
    def _():
        out_ref[:, 4::2, :] = tab + text_ref[...].reshape(BB, T, D)

    if T % BB:
        @pl.when(i == FULLTB)
        def _():
            rowmask = lax.broadcasted_iota(jnp.int32, (BB, 1, 1), 0) < (T - FULLTB * BB)
            out_ref[:, 4::2, :] = tab + jnp.where(
                rowmask, text_ref[...].reshape(BB, T, D), 0.0)

        @pl.when(i > FULLTB)
        def _():
            out_ref[:, 4::2, :] = tab
    else:
        @pl.when(i >= FULLTB)
        def _():
            out_ref[:, 4::2, :] = tab

    out_ref[:, 5::2, :] = sep_b

    age = age_ref[...]
    gen = gen_ref[...]
    at = age_t[...]
    gt = gen_t[...]
    age_row = jnp.zeros((BB, D), jnp.float32)
    for r in range(9):
        age_row = age_row + jnp.where(age == r, 1.0, 0.0) * at[r].reshape(1, D)
    gen_row = jnp.zeros((BB, D), jnp.float32)
    for r in range(3):
        gen_row = gen_row + jnp.where(gen == r, 1.0, 0.0) * gt[r].reshape(1, D)
    cls_b = jnp.broadcast_to(cls_r[...].reshape(1, D), (BB, D))
    sep_h = jnp.broadcast_to(sep_r[...].reshape(1, D), (BB, D))
    out_ref[:, 0, :] = cls_b
    out_ref[:, 1, :] = gen_row
    out_ref[:, 2, :] = age_row
    out_ref[:, 3, :] = sep_h


def _dense_pass(code_emb, num_emb, maskf, type_ids, ages2, gens2,
                age_table, gender_table, type_table, cls2, sep2, mcc2, text_tab):
    return pl.pallas_call(
        _dense_body,
        grid=(NBLK,),
        in_specs=[
            pl.BlockSpec((BB, T, D), lambda i: (i, 0, 0)),
            pl.BlockSpec((BB, T, D), lambda i: (i, 0, 0)),
            pl.BlockSpec((BB, T, 1), lambda i: (i, 0, 0)),
            pl.BlockSpec((BB, T, 1), lambda i: (i, 0, 0)),
            pl.BlockSpec((BB, 1), lambda i: (i, 0)),
            pl.BlockSpec((BB, 1), lambda i: (i, 0)),
            pl.BlockSpec((9, D), lambda i: (0, 0)),
            pl.BlockSpec((3, D), lambda i: (0, 0)),
            pl.BlockSpec((4, D), lambda i: (0, 0)),
            pl.BlockSpec((1, D), lambda i: (0, 0)),
            pl.BlockSpec((1, D), lambda i: (0, 0)),
            pl.BlockSpec((1, D), lambda i: (0, 0)),
            pl.BlockSpec((BB * T, D), lambda i: (jnp.minimum(i, NTB - 1), 0)),
        ],
        out_specs=pl.BlockSpec((BB, 404, D), lambda i: (i, 0, 0)),
        out_shape=jax.ShapeDtypeStruct((B, 404, D), jnp.float32),
    )(code_emb, num_emb, maskf, type_ids, ages2, gens2,
      age_table, gender_table, type_table, cls2, sep2, mcc2, text_tab)


NP = 20480
NCH = NP // 16
NSLOT = T * T
SCH_ALL = TROWS // 16
NW = 32
MAXTR = (SCH_ALL + NW - 1) // NW


def _dg16(x, idx):
    return lax.gather(
        x, idx[:, None],
        lax.GatherDimensionNumbers(offset_dims=(), collapsed_slice_dims=(0,),
                                   start_index_map=(0,)),
        (1,), mode=lax.GatherScatterMode.PROMISE_IN_BOUNDS)


def _make_text_sc():
    from jax.experimental.pallas import tpu_sc as plsc
    mesh = plsc.VectorSubcoreMesh(core_axis_name="c", subcore_axis_name="s")

    @functools.partial(
        pl.kernel, mesh=mesh,
        compiler_params=pltpu.CompilerParams(needs_layout_passes=False),
        out_type=jax.ShapeDtypeStruct((TROWS, D), jnp.float32),
        scratch_types=[
            pltpu.VMEM((NP,), jnp.int32),
            pltpu.VMEM((NP,), jnp.int32),
            pltpu.VMEM((TROWS,), jnp.int32),
            pltpu.VMEM((16,), jnp.int32),
            pltpu.VMEM((16, D), jnp.float32),
            pltpu.SemaphoreType.DMA,
        ],
    )
    def text_sc(loc0_hbm, loc1_hbm, text_hbm, out_hbm,
                l0_v, l1_v, win_v, tix_v, rows_v, sem):
        wid = lax.axis_index("s") * 2 + lax.axis_index("c")
        iota = lax.iota(jnp.int32, 16)
        zeros16 = jnp.zeros((16,), jnp.int32)

        pltpu.sync_copy(loc0_hbm, l0_v)
        pltpu.sync_copy(loc1_hbm, l1_v)

        @pl.loop(0, SCH_ALL)
        def _(i):
            win_v[pl.ds(i * 16, 16)] = zeros16

        @pl.loop(0, NCH)
        def _(i):
            l0 = l0_v[pl.ds(i * 16, 16)]
            l1 = l1_v[pl.ds(i * 16, 16)]
            lin = l0 * T + l1
            k1 = i * 16 + iota + 1
            slin, sk = plsc.sort_key_val(lin, k1)
            for sh in (1, 2, 4, 8):
                pl_lin = _dg16(slin, jnp.maximum(iota - sh, 0))
                pk = _dg16(sk, jnp.maximum(iota - sh, 0))
                sk = jnp.where((pl_lin == slin) & (iota >= sh),
                               jnp.maximum(sk, pk), sk)
            nlin = _dg16(slin, jnp.minimum(iota + 1, 15))
            m = ((slin != nlin) | (iota == 15)) & (slin >= 0)
            addr = jnp.maximum(slin, 0)
            cur = plsc.load_gather(win_v, [addr], mask=m)
            plsc.store_scatter(win_v, [addr], jnp.maximum(cur, sk), mask=m)

        @pl.loop(0, MAXTR)
        def _(j):
            c = wid + j * NW

            @pl.when(c < SCH_ALL)
            def _():
                w16 = win_v[pl.ds(c * 16, 16)]
                tix_v[...] = jnp.where(w16 > 0, w16 - 1, NT + iota)
                pltpu.async_copy(text_hbm.at[tix_v], rows_v, sem).wait()
                pltpu.sync_copy(rows_v, out_hbm.at[pl.ds(c * 16, 16)])

    return text_sc


_text_sc = _make_text_sc()


def _text_tab_sc(text_emb_agg, text_locations):
    loc0 = jnp.full((NP,), -1, jnp.int32).at[:NT].set(text_locations[:, 0])
    loc1 = jnp.full((NP,), -1, jnp.int32).at[:NT].set(text_locations[:, 1])
    text_pad = jnp.concatenate(
        [text_emb_agg, jnp.zeros((16, D), jnp.float32)], axis=0)
    return _text_sc(loc0, loc1, text_pad)


@jax.jit
def kernel(code_emb, num_emb, mcc_mask_positions, text_emb_agg, text_locations,
           type_ids, exam_ages, exam_genders,
           age_table, gender_table, type_table, cls_emb, sep_emb, mcc_mask_emb):
    maskf = mcc_mask_positions.astype(jnp.float32).reshape(B, T, 1)
    text_tab = _text_tab_sc(text_emb_agg, text_locations)
    final = _dense_pass(
        code_emb, num_emb, maskf, type_ids.reshape(B, T, 1),
        exam_ages.reshape(B, 1), exam_genders.reshape(B, 1),
        age_table, gender_table, type_table,
        cls_emb.reshape(1, D), sep_emb.reshape(1, D), mcc_mask_emb.reshape(1, D),
        text_tab)
    final_mask = jnp.ones((B, 2 * 202), jnp.float32)
    return final, final_mask


# speedup vs baseline: 1.1536x; 1.1536x over previous
import functools
import jax
import jax.numpy as jnp
from jax import lax
from jax.experimental import pallas as pl
from jax.experimental.pallas import tpu as pltpu

B, T, D, NT = 1024, 200, 128, 20000
BB = 16
NBLK = B // BB
NTB = -(-T // BB)
FULLTB = T // BB
TROWS = NTB * BB * T


def _dense_body(code_ref, num_ref, mask_ref, tid_ref, age_ref, gen_ref,
                age_t, gen_t, typ_t, cls_r, sep_r, mcc_r, text_ref, out_ref):
    i = pl.program_id(0)
    code = code_ref[...]
    num = num_ref[...]
    m = mask_ref[...]
    mcc = mcc_r[...].reshape(1, 1, D)
    num = jnp.where(m != 0.0, mcc, num)
    tid = tid_ref[...]
    typ = typ_t[...]
    te = jnp.where(tid == 0, typ[0].reshape(1, 1, D),
         jnp.where(tid == 1, typ[1].reshape(1, 1, D),
         jnp.where(tid == 2, typ[2].reshape(1, 1, D),
                   typ[3].reshape(1, 1, D))))
    tab = code + num + te

    sep = sep_r[...].reshape(1, 1, D)
    sep_b = jnp.broadcast_to(sep, (BB, T, D))

    @pl.when(i < FULLTB)
    def _():
        out_ref[:, 4::2, :] = tab + text_ref[...].reshape(BB, T, D)

    if T % BB:
        @pl.when(i == FULLTB)
        def _():
            rowmask = lax.broadcasted_iota(jnp.int32, (BB, 1, 1), 0) < (T - FULLTB * BB)
            out_ref[:, 4::2, :] = tab + jnp.where(
                rowmask, text_ref[...].reshape(BB, T, D), 0.0)

        @pl.when(i > FULLTB)
        def _():
            out_ref[:, 4::2, :] = tab
    else:
        @pl.when(i >= FULLTB)
        def _():
            out_ref[:, 4::2, :] = tab

    out_ref[:, 5::2, :] = sep_b

    age = age_ref[...]
    gen = gen_ref[...]
    at = age_t[...]
    gt = gen_t[...]
    age_row = jnp.zeros((BB, D), jnp.float32)
    for r in range(9):
        age_row = age_row + jnp.where(age == r, 1.0, 0.0) * at[r].reshape(1, D)
    gen_row = jnp.zeros((BB, D), jnp.float32)
    for r in range(3):
        gen_row = gen_row + jnp.where(gen == r, 1.0, 0.0) * gt[r].reshape(1, D)
    cls_b = jnp.broadcast_to(cls_r[...].reshape(1, D), (BB, D))
    sep_h = jnp.broadcast_to(sep_r[...].reshape(1, D), (BB, D))
    out_ref[:, 0, :] = cls_b
    out_ref[:, 1, :] = gen_row
    out_ref[:, 2, :] = age_row
    out_ref[:, 3, :] = sep_h


def _dense_pass(code_emb, num_emb, maskf, type_ids, ages2, gens2,
                age_table, gender_table, type_table, cls2, sep2, mcc2, text_tab):
    return pl.pallas_call(
        _dense_body,
        grid=(NBLK,),
        in_specs=[
            pl.BlockSpec((BB, T, D), lambda i: (i, 0, 0)),
            pl.BlockSpec((BB, T, D), lambda i: (i, 0, 0)),
            pl.BlockSpec((BB, T, 1), lambda i: (i, 0, 0)),
            pl.BlockSpec((BB, T, 1), lambda i: (i, 0, 0)),
            pl.BlockSpec((BB, 1), lambda i: (i, 0)),
            pl.BlockSpec((BB, 1), lambda i: (i, 0)),
            pl.BlockSpec((9, D), lambda i: (0, 0)),
            pl.BlockSpec((3, D), lambda i: (0, 0)),
            pl.BlockSpec((4, D), lambda i: (0, 0)),
            pl.BlockSpec((1, D), lambda i: (0, 0)),
            pl.BlockSpec((1, D), lambda i: (0, 0)),
            pl.BlockSpec((1, D), lambda i: (0, 0)),
            pl.BlockSpec((BB * T, D), lambda i: (jnp.minimum(i, NTB - 1), 0)),
        ],
        out_specs=pl.BlockSpec((BB, 404, D), lambda i: (i, 0, 0)),
        out_shape=jax.ShapeDtypeStruct((B, 404, D), jnp.float32),
    )(code_emb, num_emb, maskf, type_ids, ages2, gens2,
      age_table, gender_table, type_table, cls2, sep2, mcc2, text_tab)


NP = 20480
NCH = NP // 16
NSLOT = T * T
NW = 32
GPW = 24
NGRP = GPW * NW
OROWS = NGRP * 64
SCH_ALL = OROWS // 16


def _dg16(x, idx):
    return lax.gather(
        x, idx[:, None],
        lax.GatherDimensionNumbers(offset_dims=(), collapsed_slice_dims=(0,),
                                   start_index_map=(0,)),
        (1,), mode=lax.GatherScatterMode.PROMISE_IN_BOUNDS)


def _make_text_sc():
    from jax.experimental.pallas import tpu_sc as plsc
    mesh = plsc.VectorSubcoreMesh(core_axis_name="c", subcore_axis_name="s")

    @functools.partial(
        pl.kernel, mesh=mesh,
        compiler_params=pltpu.CompilerParams(needs_layout_passes=False),
        out_type=jax.ShapeDtypeStruct((OROWS, D), jnp.float32),
        scratch_types=[
            pltpu.VMEM((NP,), jnp.int32),
            pltpu.VMEM((NP,), jnp.int32),
            pltpu.VMEM((OROWS,), jnp.int32),
            pltpu.VMEM((3, 64), jnp.int32),
            pltpu.VMEM((3, 64, D), jnp.float32),
            pltpu.SemaphoreType.DMA,
            pltpu.SemaphoreType.DMA,
            pltpu.SemaphoreType.DMA,
            pltpu.SemaphoreType.DMA,
            pltpu.SemaphoreType.DMA,
            pltpu.SemaphoreType.DMA,
        ],
    )
    def text_sc(loc0_hbm, loc1_hbm, text_hbm, out_hbm,
                l0_v, l1_v, win_v, tix_v, rows_v,
                sg0, sg1, sg2, ss0, ss1, ss2):
        sg = [sg0, sg1, sg2]
        ss = [ss0, ss1, ss2]
        wid = lax.axis_index("s") * 2 + lax.axis_index("c")
        iota = lax.iota(jnp.int32, 16)
        zeros16 = jnp.zeros((16,), jnp.int32)

        pltpu.sync_copy(loc0_hbm, l0_v)
        pltpu.sync_copy(loc1_hbm, l1_v)

        @pl.loop(0, SCH_ALL)
        def _(i):
            win_v[pl.ds(i * 16, 16)] = zeros16

        @pl.loop(0, NCH)
        def _(i):
            l0 = l0_v[pl.ds(i * 16, 16)]
            l1 = l1_v[pl.ds(i * 16, 16)]
            lin = l0 * T + l1
            k1 = i * 16 + iota + 1
            slin, sk = plsc.sort_key_val(lin, k1)
            for sh in (1, 2, 4, 8):
                pl_lin = _dg16(slin, jnp.maximum(iota - sh, 0))
                pk = _dg16(sk, jnp.maximum(iota - sh, 0))
                sk = jnp.where((pl_lin == slin) & (iota >= sh),
                               jnp.maximum(sk, pk), sk)
            nlin = _dg16(slin, jnp.minimum(iota + 1, 15))
            m = ((slin != nlin) | (iota == 15)) & (slin >= 0)
            addr = jnp.maximum(slin, 0)
            cur = plsc.load_gather(win_v, [addr], mask=m)
            plsc.store_scatter(win_v, [addr], jnp.maximum(cur, sk), mask=m)

        def build_tix(t, b):
            base = (wid + t * NW) * 64
            for q in range(4):
                w16 = win_v[pl.ds(base + q * 16, 16)]
                tix_v[b, pl.ds(q * 16, 16)] = jnp.where(
                    w16 > 0, w16 - 1, NT + wid * 4 + q)

        def start_gather(t, b):
            build_tix(t, b)
            pltpu.async_copy(text_hbm.at[tix_v.at[b]], rows_v.at[b], sg[b])

        def wait_gather(b):
            pltpu.make_async_copy(text_hbm.at[tix_v.at[b]], rows_v.at[b],
                                  sg[b]).wait()

        def start_store(t, b):
            g = wid + t * NW
            pltpu.async_copy(rows_v.at[b], out_hbm.at[pl.ds(g * 64, 64)], ss[b])

        def wait_store(t, b):
            g = wid + t * NW
            pltpu.make_async_copy(rows_v.at[b], out_hbm.at[pl.ds(g * 64, 64)],
                                  ss[b]).wait()

        start_gather(0, 0)
        start_gather(1, 1)

        @pl.loop(0, GPW // 3)
        def _(j):
            for s in range(3):
                t = j * 3 + s
                b = s
                b2 = (s + 2) % 3
                wait_gather(b)
                start_store(t, b)

                @pl.when(t + 2 < GPW)
                def _():
                    @pl.when(t >= 1)
                    def _():
                        wait_store(t - 1, b2)
                    start_gather(t + 2, b2)

        for b, t in ((0, GPW - 3), (1, GPW - 2), (2, GPW - 1)):
            wait_store(t, b)

    return text_sc


_text_sc = _make_text_sc()


def _text_tab_sc(text_emb_agg, text_locations):
    loc0 = jnp.full((NP,), -1, jnp.int32).at[:NT].set(text_locations[:, 0])
    loc1 = jnp.full((NP,), -1, jnp.int32).at[:NT].set(text_locations[:, 1])
    text_pad = jnp.concatenate(
        [text_emb_agg, jnp.zeros((128, D), jnp.float32)], axis=0)
    return _text_sc(loc0, loc1, text_pad)


@jax.jit
def kernel(code_emb, num_emb, mcc_mask_positions, text_emb_agg, text_locations,
           type_ids, exam_ages, exam_genders,
           age_table, gender_table, type_table, cls_emb, sep_emb, mcc_mask_emb):
    maskf = mcc_mask_positions.astype(jnp.float32).reshape(B, T, 1)
    text_tab = _text_tab_sc(text_emb_agg, text_locations)
    final = _dense_pass(
        code_emb, num_emb, maskf, type_ids.reshape(B, T, 1),
        exam_ages.reshape(B, 1), exam_genders.reshape(B, 1),
        age_table, gender_table, type_table,
        cls_emb.reshape(1, D), sep_emb.reshape(1, D), mcc_mask_emb.reshape(1, D),
        text_tab)
    final_mask = jnp.ones((B, 2 * 202), jnp.float32)
    return final, final_mask


# speedup vs baseline: 1.1539x; 1.0002x over previous
import functools
import jax
import jax.numpy as jnp
from jax import lax
from jax.experimental import pallas as pl
from jax.experimental.pallas import tpu as pltpu

B, T, D, NT = 1024, 200, 128, 20000
BB = 16
NBLK = B // BB
NTB = -(-T // BB)
FULLTB = T // BB
TROWS = NTB * BB * T


def _dense_body(code_ref, num_ref, mask_ref, tid_ref, age_ref, gen_ref,
                age_t, gen_t, typ_t, cls_r, sep_r, mcc_r, *rest):
    with_text = len(rest) == 3
    if with_text:
        text_ref, _, out_ref = rest
    else:
        (out_ref,) = rest
    i = pl.program_id(0)
    code = code_ref[...]
    num = num_ref[...]
    m = mask_ref[...]
    mcc = mcc_r[...].reshape(1, 1, D)
    num = jnp.where(m != 0.0, mcc, num)
    tid = tid_ref[...]
    typ = typ_t[...]
    te = jnp.where(tid == 0, typ[0].reshape(1, 1, D),
         jnp.where(tid == 1, typ[1].reshape(1, 1, D),
         jnp.where(tid == 2, typ[2].reshape(1, 1, D),
                   typ[3].reshape(1, 1, D))))
    tab = code + num + te

    sep = sep_r[...].reshape(1, 1, D)
    sep_b = jnp.broadcast_to(sep, (BB, T, D))

    if with_text:
        @pl.when(i < FULLTB)
        def _():
            out_ref[:, 4::2, :] = tab + text_ref[...].reshape(BB, T, D)

        if T % BB:
            @pl.when(i == FULLTB)
            def _():
                rowmask = lax.broadcasted_iota(jnp.int32, (BB, 1, 1), 0) < (T - FULLTB * BB)
                out_ref[:, 4::2, :] = tab + jnp.where(
                    rowmask, text_ref[...].reshape(BB, T, D), 0.0)
    else:
        out_ref[:, 4::2, :] = tab

    out_ref[:, 5::2, :] = sep_b

    age = age_ref[...]
    gen = gen_ref[...]
    at = age_t[...]
    gt = gen_t[...]
    age_row = jnp.zeros((BB, D), jnp.float32)
    for r in range(9):
        age_row = age_row + jnp.where(age == r, 1.0, 0.0) * at[r].reshape(1, D)
    gen_row = jnp.zeros((BB, D), jnp.float32)
    for r in range(3):
        gen_row = gen_row + jnp.where(gen == r, 1.0, 0.0) * gt[r].reshape(1, D)
    cls_b = jnp.broadcast_to(cls_r[...].reshape(1, D), (BB, D))
    sep_h = jnp.broadcast_to(sep_r[...].reshape(1, D), (BB, D))
    out_ref[:, 0, :] = cls_b
    out_ref[:, 1, :] = gen_row
    out_ref[:, 2, :] = age_row
    out_ref[:, 3, :] = sep_h


def _dense_pass(code_emb, num_emb, maskf, type_ids, ages2, gens2,
                age_table, gender_table, type_table, cls2, sep2, mcc2, text_tab):
    def specs(off):
        return [
            pl.BlockSpec((BB, T, D), lambda i: (i + off, 0, 0)),
            pl.BlockSpec((BB, T, D), lambda i: (i + off, 0, 0)),
            pl.BlockSpec((BB, T, 1), lambda i: (i + off, 0, 0)),
            pl.BlockSpec((BB, T, 1), lambda i: (i + off, 0, 0)),
            pl.BlockSpec((BB, 1), lambda i: (i + off, 0)),
            pl.BlockSpec((BB, 1), lambda i: (i + off, 0)),
            pl.BlockSpec((9, D), lambda i: (0, 0)),
            pl.BlockSpec((3, D), lambda i: (0, 0)),
            pl.BlockSpec((4, D), lambda i: (0, 0)),
            pl.BlockSpec((1, D), lambda i: (0, 0)),
            pl.BlockSpec((1, D), lambda i: (0, 0)),
            pl.BlockSpec((1, D), lambda i: (0, 0)),
        ]

    args = (code_emb, num_emb, maskf, type_ids, ages2, gens2,
            age_table, gender_table, type_table, cls2, sep2, mcc2)

    tail = pl.pallas_call(
        _dense_body,
        grid=(NBLK - NTB,),
        in_specs=specs(NTB),
        out_specs=pl.BlockSpec((BB, 404, D), lambda i: (i + NTB, 0, 0)),
        out_shape=jax.ShapeDtypeStruct((B, 404, D), jnp.float32),
    )(*args)

    return pl.pallas_call(
        _dense_body,
        grid=(NTB,),
        in_specs=specs(0) + [
            pl.BlockSpec((BB * T, D), lambda i: (i, 0)),
            pl.BlockSpec(memory_space=pl.ANY),
        ],
        out_specs=pl.BlockSpec((BB, 404, D), lambda i: (i, 0, 0)),
        out_shape=jax.ShapeDtypeStruct((B, 404, D), jnp.float32),
        input_output_aliases={13: 0},
    )(*args, text_tab, tail)


NP = 20480
NCH = NP // 16
NSLOT = T * T
NW = 32
GPW = 24
NGRP = GPW * NW
OROWS = NGRP * 64
SCH_ALL = OROWS // 16


def _dg16(x, idx):
    return lax.gather(
        x, idx[:, None],
        lax.GatherDimensionNumbers(offset_dims=(), collapsed_slice_dims=(0,),
                                   start_index_map=(0,)),
        (1,), mode=lax.GatherScatterMode.PROMISE_IN_BOUNDS)


def _make_text_sc():
    from jax.experimental.pallas import tpu_sc as plsc
    mesh = plsc.VectorSubcoreMesh(core_axis_name="c", subcore_axis_name="s")

    @functools.partial(
        pl.kernel, mesh=mesh,
        compiler_params=pltpu.CompilerParams(needs_layout_passes=False),
        out_type=jax.ShapeDtypeStruct((OROWS, D), jnp.float32),
        scratch_types=[
            pltpu.VMEM((NP,), jnp.int32),
            pltpu.VMEM((NP,), jnp.int32),
            pltpu.VMEM((OROWS,), jnp.int32),
            pltpu.VMEM((3, 64), jnp.int32),
            pltpu.VMEM((3, 64, D), jnp.float32),
            pltpu.SemaphoreType.DMA,
            pltpu.SemaphoreType.DMA,
            pltpu.SemaphoreType.DMA,
            pltpu.SemaphoreType.DMA,
            pltpu.SemaphoreType.DMA,
            pltpu.SemaphoreType.DMA,
        ],
    )
    def text_sc(loc0_hbm, loc1_hbm, text_hbm, out_hbm,
                l0_v, l1_v, win_v, tix_v, rows_v,
                sg0, sg1, sg2, ss0, ss1, ss2):
        sg = [sg0, sg1, sg2]
        ss = [ss0, ss1, ss2]
        wid = lax.axis_index("s") * 2 + lax.axis_index("c")
        iota = lax.iota(jnp.int32, 16)
        zeros16 = jnp.zeros((16,), jnp.int32)

        pltpu.sync_copy(loc0_hbm, l0_v)
        pltpu.sync_copy(loc1_hbm, l1_v)

        @pl.loop(0, SCH_ALL)
        def _(i):
            win_v[pl.ds(i * 16, 16)] = zeros16

        @pl.loop(0, NCH)
        def _(i):
            l0 = l0_v[pl.ds(i * 16, 16)]
            l1 = l1_v[pl.ds(i * 16, 16)]
            lin = l0 * T + l1
            k1 = i * 16 + iota + 1
            slin, sk = plsc.sort_key_val(lin, k1)
            for sh in (1, 2, 4, 8):
                pl_lin = _dg16(slin, jnp.maximum(iota - sh, 0))
                pk = _dg16(sk, jnp.maximum(iota - sh, 0))
                sk = jnp.where((pl_lin == slin) & (iota >= sh),
                               jnp.maximum(sk, pk), sk)
            nlin = _dg16(slin, jnp.minimum(iota + 1, 15))
            m = ((slin != nlin) | (iota == 15)) & (slin >= 0)
            addr = jnp.maximum(slin, 0)
            cur = plsc.load_gather(win_v, [addr], mask=m)
            plsc.store_scatter(win_v, [addr], jnp.maximum(cur, sk), mask=m)

        def build_tix(t, b):
            base = (wid + t * NW) * 64
            for q in range(4):
                w16 = win_v[pl.ds(base + q * 16, 16)]
                tix_v[b, pl.ds(q * 16, 16)] = jnp.where(
                    w16 > 0, w16 - 1, NT + wid * 4 + q)

        def start_gather(t, b):
            build_tix(t, b)
            pltpu.async_copy(text_hbm.at[tix_v.at[b]], rows_v.at[b], sg[b])

        def wait_gather(b):
            pltpu.make_async_copy(text_hbm.at[tix_v.at[b]], rows_v.at[b],
                                  sg[b]).wait()

        def start_store(t, b):
            g = wid + t * NW
            pltpu.async_copy(rows_v.at[b], out_hbm.at[pl.ds(g * 64, 64)], ss[b])

        def wait_store(t, b):
            g = wid + t * NW
            pltpu.make_async_copy(rows_v.at[b], out_hbm.at[pl.ds(g * 64, 64)],
                                  ss[b]).wait()

        start_gather(0, 0)
        start_gather(1, 1)

        @pl.loop(0, GPW // 3)
        def _(j):
            for s in range(3):
                t = j * 3 + s
                b = s
                b2 = (s + 2) % 3
                wait_gather(b)
                start_store(t, b)

                @pl.when(t + 2 < GPW)
                def _():
                    @pl.when(t >= 1)
                    def _():
                        wait_store(t - 1, b2)
                    start_gather(t + 2, b2)

        for b, t in ((0, GPW - 3), (1, GPW - 2), (2, GPW - 1)):
            wait_store(t, b)

    return text_sc


_text_sc = _make_text_sc()


def _text_tab_sc(text_emb_agg, text_locations):
    loc0 = jnp.full((NP,), -1, jnp.int32).at[:NT].set(text_locations[:, 0])
    loc1 = jnp.full((NP,), -1, jnp.int32).at[:NT].set(text_locations[:, 1])
    text_pad = jnp.concatenate(
        [text_emb_agg, jnp.zeros((128, D), jnp.float32)], axis=0)
    return _text_sc(loc0, loc1, text_pad)


@jax.jit
def kernel(code_emb, num_emb, mcc_mask_positions, text_emb_agg, text_locations,
           type_ids, exam_ages, exam_genders,
           age_table, gender_table, type_table, cls_emb, sep_emb, mcc_mask_emb):
    maskf = mcc_mask_positions.astype(jnp.float32).reshape(B, T, 1)
    text_tab = _text_tab_sc(text_emb_agg, text_locations)
    final = _dense_pass(
        code_emb, num_emb, maskf, type_ids.reshape(B, T, 1),
        exam_ages.reshape(B, 1), exam_genders.reshape(B, 1),
        age_table, gender_table, type_table,
        cls_emb.reshape(1, D), sep_emb.reshape(1, D), mcc_mask_emb.reshape(1, D),
        text_tab)
    final_mask = jnp.ones((B, 2 * 202), jnp.float32)
    return final, final_mask
